# baseline (speedup 1.0000x reference)
import functools

import jax
import jax.numpy as jnp
from jax import lax
from jax.experimental import pallas as pl
from jax.experimental.pallas import tpu as pltpu
from jax.experimental.pallas import tpu_sc as plsc

D_MODEL = 64
SCALE = 8.0

GRP = 128
CHUNK_GRPS = 4
CHUNK = GRP * CHUNK_GRPS


@jax.jit
def _embed_sc(table, x2d):
    info = plsc.get_sparse_core_info()
    nw = info.num_cores * info.num_subcores
    n_groups = x2d.shape[0]
    b_total = n_groups * GRP
    grps_per_w = n_groups // nw
    chunks_per_w = grps_per_w // CHUNK_GRPS

    mesh = plsc.VectorSubcoreMesh(core_axis_name="c", subcore_axis_name="s")

    @functools.partial(
        pl.kernel,
        mesh=mesh,
        out_type=jax.ShapeDtypeStruct((b_total, D_MODEL), jnp.float32),
        scratch_types=[
            pltpu.VMEM((CHUNK_GRPS, GRP), jnp.int32),
            pltpu.VMEM((CHUNK, D_MODEL), jnp.float32),
            pltpu.SemaphoreType.DMA,
        ],
        compiler_params=pltpu.CompilerParams(use_tc_tiling_on_sc=False),
    )
    def k(table_hbm, x_hbm, out_hbm, idx_v, rows_v, sem):
        wid = lax.axis_index("s") * info.num_cores + lax.axis_index("c")
        g_base = wid * grps_per_w

        def chunk_body(ci, carry):
            g0 = g_base + ci * CHUNK_GRPS
            row0 = g0 * GRP
            pltpu.sync_copy(x_hbm.at[pl.ds(g0, CHUNK_GRPS)], idx_v)
            cps = [
                pltpu.async_copy(
                    table_hbm.at[idx_v.at[j]],
                    rows_v.at[pl.ds(j * GRP, GRP)],
                    sem,
                )
                for j in range(CHUNK_GRPS)
            ]
            for cp in cps:
                cp.wait()

            def scale_body(r, c2):
                for j in range(D_MODEL // 16):
                    rows_v[r, pl.ds(j * 16, 16)] = (
                        rows_v[r, pl.ds(j * 16, 16)] * SCALE
                    )
                return c2

            lax.fori_loop(0, CHUNK, scale_body, 0, unroll=2)

            pltpu.sync_copy(rows_v, out_hbm.at[pl.ds(row0, CHUNK)])
            return carry

        lax.fori_loop(0, chunks_per_w, chunk_body, 0)

    return k(table, x2d)


def kernel(x, table):
    b_total = x.shape[0] * x.shape[1]
    x_flat = x.reshape(b_total).astype(jnp.int32)
    x2d = x_flat.reshape(b_total // GRP, GRP)
    out = _embed_sc(table, x2d)
    return out.reshape(x.shape[0], x.shape[1], D_MODEL)
